# baseline (speedup 1.0000x reference)
import functools
import math

import jax
import jax.numpy as jnp
import numpy as np
from jax import lax
from jax.experimental import pallas as pl
from jax.experimental.pallas import tpu as pltpu
from jax.experimental.pallas import tpu_sc as plsc

N = 32768
D = 3
T = 16
RESOLUTION = 64
RADIUS = 1.0
SCALE = 8.0
MAX_CHANNELS = 4
BATCH_LEN = 16
NUM_SEG = BATCH_LEN * MAX_CHANNELS
ROW = RESOLUTION * T
ACC = NUM_SEG * ROW

SCN = 8192
TCN = N - SCN

NC = 2
NS = 16
NW = NC * NS
PPW = SCN // NW

_LIN = np.linspace(-RADIUS, RADIUS, RESOLUTION).astype(np.float64)
_AVALS = [float(a) for a in np.exp(-SCALE * _LIN)]

_mesh = plsc.VectorSubcoreMesh(core_axis_name="c", subcore_axis_name="s")


@functools.partial(
    pl.kernel,
    mesh=_mesh,
    out_type=jax.ShapeDtypeStruct((NW, ACC), jnp.float32),
    scratch_types=[
        pltpu.VMEM((PPW * D + 16,), jnp.float32),
        pltpu.VMEM((PPW + 16,), jnp.int32),
        pltpu.VMEM((PPW + 16,), jnp.int32),
        pltpu.VMEM((D, T), jnp.float32),
        pltpu.VMEM((ACC,), jnp.float32),
    ],
)
def _sc_partials(xf_hbm, v_hbm, ind_hbm, ch_hbm, out_hbm,
                 x_v, ind_v, ch_v, v_v, acc_v):
    wid = lax.axis_index("s") * NC + lax.axis_index("c")
    base = wid * PPW

    pltpu.sync_copy(xf_hbm.at[pl.ds(base * D, PPW * D)],
                    x_v.at[pl.ds(0, PPW * D)])
    pltpu.sync_copy(ind_hbm.at[pl.ds(base, PPW)], ind_v.at[pl.ds(0, PPW)])
    pltpu.sync_copy(ch_hbm.at[pl.ds(base, PPW)], ch_v.at[pl.ds(0, PPW)])
    pltpu.sync_copy(v_hbm, v_v)

    zero16 = jnp.zeros((16,), jnp.float32)

    def zero_body(i, carry):
        acc_v[pl.ds(i * 16, 16)] = zero16
        return carry

    lax.fori_loop(0, ACC // 16, zero_body, 0)

    v0 = v_v[0]
    v1 = v_v[1]
    v2 = v_v[2]

    def point_body(i, carry):
        xv = x_v[pl.ds(i * D, 16)]
        nh = xv[0] * v0 + xv[1] * v1 + xv[2] * v2
        e = jnp.exp(SCALE * nh)
        seg = (MAX_CHANNELS * ind_v[pl.ds(i, 16)][0]
               + ch_v[pl.ds(i, 16)][0])
        rowbase = seg * ROW
        for r in range(RESOLUTION):
            sig = 1.0 / (1.0 + e * _AVALS[r])
            off = rowbase + r * T
            plsc.addupdate(acc_v.at[pl.ds(off, 16)], sig)
        return carry

    lax.fori_loop(0, PPW, point_body, 0)

    pltpu.sync_copy(acc_v, out_hbm.at[wid])


CHUNK = 4096
NUM_BLOCKS = TCN // CHUNK

_A = SCALE * math.log2(math.e)
_CLAMP = 126.0

_S = (np.arange(T)[:, None] == (np.arange(T * RESOLUTION)[None, :] % T)).astype(
    np.float64
) * np.exp2(-_A * _LIN)[np.arange(T * RESOLUTION) // T][None, :]
_S = _S.astype(np.float32)


def _tc_kernel(x_ref, v_ref, s_ref, index_ref, chan_ref, out_ref):
    step = pl.program_id(0)

    x = x_ref[...]
    v2 = _A * v_ref[...]
    m = jnp.dot(x, v2, preferred_element_type=jnp.float32)
    m = jnp.clip(m, -_CLAMP, _CLAMP)
    e = jnp.exp2(m).astype(jnp.bfloat16)

    p = jnp.dot(e, s_ref[...], preferred_element_type=jnp.float32)
    sigb = (1.0 / (1.0 + p)).astype(jnp.bfloat16)

    idx = MAX_CHANNELS * index_ref[0] + chan_ref[0]
    seg = jax.lax.broadcasted_iota(jnp.int32, (NUM_SEG, CHUNK), 0)
    onehot = (idx == seg).astype(jnp.bfloat16)

    contrib = jnp.dot(onehot, sigb, preferred_element_type=jnp.float32)

    @pl.when(step == 0)
    def _init():
        out_ref[...] = contrib

    @pl.when(step > 0)
    def _acc():
        out_ref[...] = out_ref[...] + contrib


def _combine_kernel(sc_ref, tc_ref, out_ref):
    acc = jnp.sum(sc_ref[...], axis=0) + tc_ref[...]
    mx = jnp.max(acc, axis=1, keepdims=True)
    mx = jnp.where(mx == 0.0, 1.0, mx)
    out_ref[...] = acc / mx


@jax.jit
def kernel(x, v, index, channels):
    index = index.astype(jnp.int32)
    channels = channels.astype(jnp.int32)

    xf = x.reshape(N * D)
    sc_parts = _sc_partials(xf, v, index, channels)
    sc_parts = sc_parts.reshape(NW, NUM_SEG, ROW)

    x_tc = x[SCN:]
    index3 = index[SCN:].reshape(NUM_BLOCKS, 1, CHUNK)
    chan3 = channels[SCN:].reshape(NUM_BLOCKS, 1, CHUNK)
    s = jnp.asarray(_S, dtype=jnp.bfloat16)

    tc_part = pl.pallas_call(
        _tc_kernel,
        grid=(NUM_BLOCKS,),
        in_specs=[
            pl.BlockSpec((CHUNK, D), lambda i: (i, 0)),
            pl.BlockSpec((D, T), lambda i: (0, 0)),
            pl.BlockSpec((T, T * RESOLUTION), lambda i: (0, 0)),
            pl.BlockSpec((1, 1, CHUNK), lambda i: (i, 0, 0)),
            pl.BlockSpec((1, 1, CHUNK), lambda i: (i, 0, 0)),
        ],
        out_specs=pl.BlockSpec((NUM_SEG, ROW), lambda i: (0, 0)),
        out_shape=jax.ShapeDtypeStruct((NUM_SEG, ROW), jnp.float32),
    )(x_tc, v, s, index3, chan3)

    out = pl.pallas_call(
        _combine_kernel,
        grid=(1,),
        in_specs=[
            pl.BlockSpec((NW, NUM_SEG, ROW), lambda i: (0, 0, 0)),
            pl.BlockSpec((NUM_SEG, ROW), lambda i: (0, 0)),
        ],
        out_specs=pl.BlockSpec((NUM_SEG, ROW), lambda i: (0, 0)),
        out_shape=jax.ShapeDtypeStruct((NUM_SEG, ROW), jnp.float32),
    )(sc_parts, tc_part)

    return out.reshape(BATCH_LEN, MAX_CHANNELS, RESOLUTION, T)


# speedup vs baseline: 1.2040x; 1.2040x over previous
import functools
import math

import jax
import jax.numpy as jnp
import numpy as np
from jax import lax
from jax.experimental import pallas as pl
from jax.experimental.pallas import tpu as pltpu
from jax.experimental.pallas import tpu_sc as plsc

N = 32768
D = 3
T = 16
RESOLUTION = 64
RADIUS = 1.0
SCALE = 8.0
MAX_CHANNELS = 4
BATCH_LEN = 16
NUM_SEG = BATCH_LEN * MAX_CHANNELS
ROW = RESOLUTION * T
ACC = NUM_SEG * ROW

SCN = 8192
TCN = N - SCN

NC = 2
NS = 16
NW = NC * NS
PPW = SCN // NW

_LIN = np.linspace(-RADIUS, RADIUS, RESOLUTION).astype(np.float64)
_AVALS = [float(a) for a in np.exp(-SCALE * _LIN)]

_mesh = plsc.VectorSubcoreMesh(core_axis_name="c", subcore_axis_name="s")


@functools.partial(
    pl.kernel,
    mesh=_mesh,
    out_type=jax.ShapeDtypeStruct((NW, ACC), jnp.float32),
    scratch_types=[
        pltpu.VMEM((PPW + 16,), jnp.float32),
        pltpu.VMEM((PPW + 16,), jnp.float32),
        pltpu.VMEM((PPW + 16,), jnp.float32),
        pltpu.VMEM((PPW + 16,), jnp.int32),
        pltpu.VMEM((PPW + 16,), jnp.int32),
        pltpu.VMEM((D, T), jnp.float32),
        pltpu.VMEM((ACC,), jnp.float32),
    ],
)
def _sc_partials(x0_hbm, x1_hbm, x2_hbm, v_hbm, ind_hbm, ch_hbm, out_hbm,
                 x0_v, x1_v, x2_v, ind_v, ch_v, v_v, acc_v):
    wid = lax.axis_index("s") * NC + lax.axis_index("c")
    base = wid * PPW

    pltpu.sync_copy(x0_hbm.at[pl.ds(base, PPW)], x0_v.at[pl.ds(0, PPW)])
    pltpu.sync_copy(x1_hbm.at[pl.ds(base, PPW)], x1_v.at[pl.ds(0, PPW)])
    pltpu.sync_copy(x2_hbm.at[pl.ds(base, PPW)], x2_v.at[pl.ds(0, PPW)])
    pltpu.sync_copy(ind_hbm.at[pl.ds(base, PPW)], ind_v.at[pl.ds(0, PPW)])
    pltpu.sync_copy(ch_hbm.at[pl.ds(base, PPW)], ch_v.at[pl.ds(0, PPW)])
    pltpu.sync_copy(v_hbm, v_v)

    zero16 = jnp.zeros((16,), jnp.float32)

    def zero_body(i, carry):
        acc_v[pl.ds(i * 16, 16)] = zero16
        return carry

    lax.fori_loop(0, ACC // 16, zero_body, 0)

    v0 = v_v[0]
    v1 = v_v[1]
    v2 = v_v[2]

    def point_body(i, carry):
        x0 = x0_v[pl.ds(i, 16)][0]
        x1 = x1_v[pl.ds(i, 16)][0]
        x2 = x2_v[pl.ds(i, 16)][0]
        nh = x0 * v0 + x1 * v1 + x2 * v2
        e = jnp.exp(SCALE * nh)
        seg = (MAX_CHANNELS * ind_v[pl.ds(i, 16)][0]
               + ch_v[pl.ds(i, 16)][0])
        rowbase = seg * ROW
        for r in range(RESOLUTION):
            sig = 1.0 / (1.0 + e * _AVALS[r])
            off = rowbase + r * T
            plsc.addupdate(acc_v.at[pl.ds(off, 16)], sig)
        return carry

    lax.fori_loop(0, PPW, point_body, 0)

    pltpu.sync_copy(acc_v, out_hbm.at[wid])


CHUNK = 4096
NUM_BLOCKS = TCN // CHUNK

_A = SCALE * math.log2(math.e)
_CLAMP = 126.0

_S = (np.arange(T)[:, None] == (np.arange(T * RESOLUTION)[None, :] % T)).astype(
    np.float64
) * np.exp2(-_A * _LIN)[np.arange(T * RESOLUTION) // T][None, :]
_S = _S.astype(np.float32)


def _tc_kernel(x_ref, v_ref, s_ref, index_ref, chan_ref, out_ref):
    step = pl.program_id(0)

    x = x_ref[...]
    v2 = _A * v_ref[...]
    m = jnp.dot(x, v2, preferred_element_type=jnp.float32)
    m = jnp.clip(m, -_CLAMP, _CLAMP)
    e = jnp.exp2(m).astype(jnp.bfloat16)

    p = jnp.dot(e, s_ref[...], preferred_element_type=jnp.float32)
    sigb = (1.0 / (1.0 + p)).astype(jnp.bfloat16)

    idx = MAX_CHANNELS * index_ref[0] + chan_ref[0]
    seg = jax.lax.broadcasted_iota(jnp.int32, (NUM_SEG, CHUNK), 0)
    onehot = (idx == seg).astype(jnp.bfloat16)

    contrib = jnp.dot(onehot, sigb, preferred_element_type=jnp.float32)

    @pl.when(step == 0)
    def _init():
        out_ref[...] = contrib

    @pl.when(step > 0)
    def _acc():
        out_ref[...] = out_ref[...] + contrib


def _combine_kernel(sc_ref, tc_ref, out_ref):
    acc = jnp.sum(sc_ref[...], axis=0) + tc_ref[...]
    mx = jnp.max(acc, axis=1, keepdims=True)
    mx = jnp.where(mx == 0.0, 1.0, mx)
    out_ref[...] = acc / mx


@jax.jit
def kernel(x, v, index, channels):
    index = index.astype(jnp.int32)
    channels = channels.astype(jnp.int32)

    xs = x[:SCN]
    sc_parts = _sc_partials(xs[:, 0], xs[:, 1], xs[:, 2], v, index, channels)
    sc_parts = sc_parts.reshape(NW, NUM_SEG, ROW)

    x_tc = x[SCN:]
    index3 = index[SCN:].reshape(NUM_BLOCKS, 1, CHUNK)
    chan3 = channels[SCN:].reshape(NUM_BLOCKS, 1, CHUNK)
    s = jnp.asarray(_S, dtype=jnp.bfloat16)

    tc_part = pl.pallas_call(
        _tc_kernel,
        grid=(NUM_BLOCKS,),
        in_specs=[
            pl.BlockSpec((CHUNK, D), lambda i: (i, 0)),
            pl.BlockSpec((D, T), lambda i: (0, 0)),
            pl.BlockSpec((T, T * RESOLUTION), lambda i: (0, 0)),
            pl.BlockSpec((1, 1, CHUNK), lambda i: (i, 0, 0)),
            pl.BlockSpec((1, 1, CHUNK), lambda i: (i, 0, 0)),
        ],
        out_specs=pl.BlockSpec((NUM_SEG, ROW), lambda i: (0, 0)),
        out_shape=jax.ShapeDtypeStruct((NUM_SEG, ROW), jnp.float32),
    )(x_tc, v, s, index3, chan3)

    out = pl.pallas_call(
        _combine_kernel,
        grid=(1,),
        in_specs=[
            pl.BlockSpec((NW, NUM_SEG, ROW), lambda i: (0, 0, 0)),
            pl.BlockSpec((NUM_SEG, ROW), lambda i: (0, 0)),
        ],
        out_specs=pl.BlockSpec((NUM_SEG, ROW), lambda i: (0, 0)),
        out_shape=jax.ShapeDtypeStruct((NUM_SEG, ROW), jnp.float32),
    )(sc_parts, tc_part)

    return out.reshape(BATCH_LEN, MAX_CHANNELS, RESOLUTION, T)
